# baseline (speedup 1.0000x reference)
import functools

import jax
import jax.numpy as jnp
from jax import lax
from jax.experimental import pallas as pl
from jax.experimental.pallas import tpu as pltpu
from jax.experimental.pallas import tpu_sc as plsc

_N = 1000
_E = 16000
_NG = 64
_H = 64
_GPB = 4
_COLS = _GPB * _H
_HALVES = 2
_WIDE = _COLS * _HALVES
_STEPS = _NG // (_GPB * _HALVES)
_ROWS = 32
_TILES = 32
_NPAD = 1024


def _rsqrt_newton(x):
    i = plsc.bitcast(x, jnp.int32)
    i = 0x5F3759DF - lax.shift_right_arithmetic(i, 1)
    y = plsc.bitcast(i, jnp.float32)
    for _ in range(3):
        y = y * (1.5 - 0.5 * x * y * y)
    return y


_ECH = _E // 16
_CPT = -(-_ECH // 16)
_DPS = _NPAD // 16


def _adj_body(src_hbm, dst_hbm, out_hbm, src_v, dst_v, deg_v, part_v, dinv_v,
              dloc_v, blk_v, shdeg_s, shdinv_s):
    sid = lax.axis_index("s")
    wid = sid * 2 + lax.axis_index("c")
    r0 = wid * _ROWS
    pltpu.sync_copy(src_hbm, src_v)
    pltpu.sync_copy(dst_hbm, dst_v)
    ones = jnp.ones((16,), jnp.float32)
    zeros = jnp.zeros((16,), jnp.float32)

    def init_deg(i, c):
        deg_v[pl.ds(i * 16, 16)] = zeros
        return c
    lax.fori_loop(0, _NPAD // 16, init_deg, 0, unroll=8)

    def count(i, c):
        ch = sid * _CPT + i

        @pl.when(ch < _ECH)
        def _():
            d = dst_v[pl.ds(ch * 16, 16)]
            cnt, last = plsc.scan_count(d)
            plsc.addupdate_scatter(deg_v, [d], cnt.astype(jnp.float32),
                                   mask=last)
        return c
    lax.fori_loop(0, _CPT, count, 0, unroll=8)

    pltpu.sync_copy(deg_v, shdeg_s.at[pl.ds(sid * _NPAD, _NPAD)])
    plsc.subcore_barrier()

    for r in range(16):
        pltpu.sync_copy(shdeg_s.at[pl.ds(r * _NPAD + sid * _DPS, _DPS)],
                        part_v.at[pl.ds(r * _DPS, _DPS)])
    for c4 in range(_DPS // 16):
        acc = ones
        for r in range(16):
            acc = acc + part_v[pl.ds(r * _DPS + c4 * 16, 16)]
        dloc_v[pl.ds(c4 * 16, 16)] = _rsqrt_newton(acc)
    pltpu.sync_copy(dloc_v, shdinv_s.at[pl.ds(sid * _DPS, _DPS)])
    plsc.subcore_barrier()
    pltpu.sync_copy(shdinv_s, dinv_v)

    def zero_blk(i, c):
        blk_v[pl.ds(i * 16, 16)] = zeros
        return c
    lax.fori_loop(0, _ROWS * _N // 16, zero_blk, 0, unroll=16)

    def edge(i, c):
        s = src_v[pl.ds(i * 16, 16)]
        d = dst_v[pl.ds(i * 16, 16)]
        owned = (d >= r0) & (d < r0 + _ROWS)
        v = plsc.load_gather(dinv_v, [s]) * plsc.load_gather(dinv_v, [d])
        f = jnp.where(owned, (d - r0) * _N + s, 0)
        cnt, last = plsc.scan_count(f, mask=owned)
        plsc.addupdate_scatter(blk_v, [f], v * cnt.astype(jnp.float32),
                               mask=last)
        return c
    lax.fori_loop(0, _E // 16, edge, 0, unroll=8)

    iota16 = lax.iota(jnp.int32, 16)
    for c2 in range(_ROWS // 16):
        n16 = r0 + c2 * 16 + iota16
        valid = n16 < _N
        dd = plsc.load_gather(dinv_v, [n16])
        f = jnp.where(valid, (n16 - r0) * _N + n16, 0)
        plsc.addupdate_scatter(blk_v, [f], dd * dd, mask=valid)

    @pl.when(wid < _TILES - 1)
    def _copy_full():
        pltpu.sync_copy(blk_v, out_hbm.at[pl.ds(r0 * _N, _ROWS * _N)])

    @pl.when(wid == _TILES - 1)
    def _copy_tail():
        rows = _N - (_TILES - 1) * _ROWS
        pltpu.sync_copy(blk_v.at[pl.ds(0, rows * _N)],
                        out_hbm.at[pl.ds(r0 * _N, rows * _N)])


@functools.cache
def _build_adj_kernel():
    return pl.kernel(
        _adj_body,
        out_type=jax.ShapeDtypeStruct((_N * _N,), jnp.float32),
        mesh=plsc.VectorSubcoreMesh(core_axis_name="c", subcore_axis_name="s"),
        compiler_params=pltpu.CompilerParams(needs_layout_passes=False),
        scratch_types=[
            pltpu.VMEM((_E,), jnp.int32),
            pltpu.VMEM((_E,), jnp.int32),
            pltpu.VMEM((_NPAD,), jnp.float32),
            pltpu.VMEM((16 * _DPS,), jnp.float32),
            pltpu.VMEM((_NPAD,), jnp.float32),
            pltpu.VMEM((_DPS,), jnp.float32),
            pltpu.VMEM((_ROWS * _N,), jnp.float32),
            pltpu.VMEM_SHARED((16 * _NPAD,), jnp.float32),
            pltpu.VMEM_SHARED((_NPAD,), jnp.float32),
        ],
    )


def _gcn_body(xt_ref, A_ref, wk_ref, mavg_ref, vec_ref, out_ref):
    A = A_ref[...]
    M = mavg_ref[...]
    halves = [xt_ref[:, k * _COLS:(k + 1) * _COLS].astype(jnp.float32)
              for k in range(_HALVES)]
    for li in range(3):
        wk = wk_ref[li]
        bias = vec_ref[3 * li:3 * li + 1, :]
        g = vec_ref[3 * li + 1:3 * li + 2, :]
        b = vec_ref[3 * li + 2:3 * li + 3, :]
        hl = jnp.concatenate(
            [jax.lax.dot(h, wk, preferred_element_type=jnp.float32)
             for h in halves], axis=1)
        ha_w = jax.lax.dot(A, hl, preferred_element_type=jnp.float32)
        nxt = []
        for k in range(_HALVES):
            ha = ha_w[:, k * _COLS:(k + 1) * _COLS] + bias
            m = jax.lax.dot(ha, M, preferred_element_type=jnp.float32)
            q = jax.lax.dot(ha * ha, M, preferred_element_type=jnp.float32)
            o = jnp.maximum((ha - m) * lax.rsqrt(q - m * m + 1e-5) * g + b,
                            0.0)
            nxt.append(o if li == 0 else o + halves[k])
        halves = nxt
    out = jnp.concatenate(
        [jnp.mean(h, axis=0, keepdims=True) for h in halves], axis=1)
    out_ref[...] = out.reshape(1, 1, _WIDE)


def _tail_body(seq_ref, P_ref, w1_ref, b1_ref, w2_ref, b2_ref, out_ref):
    pooled = jax.lax.dot(P_ref[...], seq_ref[...],
                         preferred_element_type=jnp.float32)
    hm = jnp.maximum(jax.lax.dot(pooled, w1_ref[...],
                                 preferred_element_type=jnp.float32)
                     + b1_ref[...], 0.0)
    out_ref[...] = jax.lax.dot(hm, w2_ref[...],
                               preferred_element_type=jnp.float32) + b2_ref[...]


def _block_diag(W, k):
    n, m = W.shape
    out = jnp.zeros((k * n, k * m), W.dtype)
    for i in range(k):
        out = out.at[i * n:(i + 1) * n, i * m:(i + 1) * m].set(W)
    return out


def kernel(x, base_adj, base_edge_index, W_in, b_in, ln_in_g, ln_in_b,
           W_h0, b_h0, ln_h0_g, ln_h0_b, W_h1, b_h1, ln_h1_g, ln_h1_b,
           W_c1, b_c1, W_c2, b_c2):
    del base_adj
    src = base_edge_index[0]
    dst = base_edge_index[1]
    A = _build_adj_kernel()(src, dst).reshape(_N, _N)

    xt = jnp.transpose(x.reshape(_NG, _N, _H).astype(jnp.bfloat16),
                       (1, 0, 2)).reshape(_N, _NG * _H)

    wks = jnp.stack([_block_diag(W_in, _GPB), _block_diag(W_h0, _GPB),
                     _block_diag(W_h1, _GPB)])
    mavg = _block_diag(jnp.full((_H, _H), 1.0 / _H, jnp.float32), _GPB)
    vecs = jnp.zeros((16, _COLS), jnp.float32)
    for r_i, v in enumerate([b_in, ln_in_g, ln_in_b, b_h0, ln_h0_g, ln_h0_b,
                             b_h1, ln_h1_g, ln_h1_b]):
        vecs = vecs.at[r_i].set(jnp.tile(v, _GPB))

    seq = pl.pallas_call(
        _gcn_body,
        grid=(_STEPS,),
        in_specs=[
            pl.BlockSpec((_N, _WIDE), lambda i: (0, i)),
            pl.BlockSpec((_N, _N), lambda i: (0, 0)),
            pl.BlockSpec((3, _COLS, _COLS), lambda i: (0, 0, 0)),
            pl.BlockSpec((_COLS, _COLS), lambda i: (0, 0)),
            pl.BlockSpec((16, _COLS), lambda i: (0, 0)),
        ],
        out_specs=pl.BlockSpec((1, 1, _WIDE), lambda i: (i, 0, 0)),
        out_shape=jax.ShapeDtypeStruct((_STEPS, 1, _WIDE), jnp.float32),
        compiler_params=pltpu.CompilerParams(
            dimension_semantics=("arbitrary",)),
    )(xt, A, wks, mavg, vecs)

    seq = seq.reshape(_NG, _H)
    P = jnp.repeat(jnp.eye(8, dtype=jnp.float32), 8, axis=1) / 8.0
    logits = pl.pallas_call(
        _tail_body,
        out_shape=jax.ShapeDtypeStruct((8, 2), jnp.float32),
    )(seq, P, W_c1, b_c1.reshape(1, _H), W_c2, b_c2.reshape(1, 2))
    return logits


# speedup vs baseline: 1.0837x; 1.0837x over previous
import functools

import jax
import jax.numpy as jnp
from jax import lax
from jax.experimental import pallas as pl
from jax.experimental.pallas import tpu as pltpu
from jax.experimental.pallas import tpu_sc as plsc

_N = 1000
_E = 16000
_NG = 64
_H = 64
_GPB = 4
_COLS = _GPB * _H
_HALVES = 2
_WIDE = _COLS * _HALVES
_STEPS = _NG // (_GPB * _HALVES)
_ROWS = 32
_TILES = 32
_NPAD = 1024


def _rsqrt_newton(x):
    i = plsc.bitcast(x, jnp.int32)
    i = 0x5F3759DF - lax.shift_right_arithmetic(i, 1)
    y = plsc.bitcast(i, jnp.float32)
    for _ in range(3):
        y = y * (1.5 - 0.5 * x * y * y)
    return y


_ECH = _E // 16
_CPT = -(-_ECH // 16)
_DPS = _NPAD // 16


def _adj_body(src_hbm, dst_hbm, out_hbm, src_v, dst_v, deg_v, part_v, dinv_v,
              dloc_v, blk_v, shdeg_s, shdinv_s):
    sid = lax.axis_index("s")
    wid = sid * 2 + lax.axis_index("c")
    r0 = wid * _ROWS
    pltpu.sync_copy(src_hbm, src_v)
    pltpu.sync_copy(dst_hbm, dst_v)
    ones = jnp.ones((16,), jnp.float32)
    zeros = jnp.zeros((16,), jnp.float32)

    def init_deg(i, c):
        deg_v[pl.ds(i * 16, 16)] = zeros
        return c
    lax.fori_loop(0, _NPAD // 16, init_deg, 0, unroll=8)

    def count(i, c):
        ch = sid * _CPT + i

        @pl.when(ch < _ECH)
        def _():
            d = dst_v[pl.ds(ch * 16, 16)]
            cnt, last = plsc.scan_count(d)
            plsc.addupdate_scatter(deg_v, [d], cnt.astype(jnp.float32),
                                   mask=last)
        return c
    lax.fori_loop(0, _CPT, count, 0, unroll=8)

    pltpu.sync_copy(deg_v, shdeg_s.at[pl.ds(sid * _NPAD, _NPAD)])
    plsc.subcore_barrier()

    for r in range(16):
        pltpu.sync_copy(shdeg_s.at[pl.ds(r * _NPAD + sid * _DPS, _DPS)],
                        part_v.at[pl.ds(r * _DPS, _DPS)])
    for c4 in range(_DPS // 16):
        acc = ones
        for r in range(16):
            acc = acc + part_v[pl.ds(r * _DPS + c4 * 16, 16)]
        dloc_v[pl.ds(c4 * 16, 16)] = _rsqrt_newton(acc)
    pltpu.sync_copy(dloc_v, shdinv_s.at[pl.ds(sid * _DPS, _DPS)])
    plsc.subcore_barrier()
    pltpu.sync_copy(shdinv_s, dinv_v)

    def zero_blk(i, c):
        blk_v[pl.ds(i * 16, 16)] = zeros
        return c
    lax.fori_loop(0, _ROWS * _N // 16, zero_blk, 0, unroll=16)

    def edge(i, c):
        s = src_v[pl.ds(i * 16, 16)]
        d = dst_v[pl.ds(i * 16, 16)]
        owned = (d >= r0) & (d < r0 + _ROWS)
        v = plsc.load_gather(dinv_v, [s]) * plsc.load_gather(dinv_v, [d])
        f = jnp.where(owned, (d - r0) * _N + s, 0)
        cnt, last = plsc.scan_count(f, mask=owned)
        plsc.addupdate_scatter(blk_v, [f], v * cnt.astype(jnp.float32),
                               mask=last)
        return c
    lax.fori_loop(0, _E // 16, edge, 0, unroll=8)

    iota16 = lax.iota(jnp.int32, 16)
    for c2 in range(_ROWS // 16):
        n16 = r0 + c2 * 16 + iota16
        valid = n16 < _N
        dd = plsc.load_gather(dinv_v, [n16])
        f = jnp.where(valid, (n16 - r0) * _N + n16, 0)
        plsc.addupdate_scatter(blk_v, [f], dd * dd, mask=valid)

    @pl.when(wid < _TILES - 1)
    def _copy_full():
        pltpu.sync_copy(blk_v, out_hbm.at[pl.ds(r0 * _N, _ROWS * _N)])

    @pl.when(wid == _TILES - 1)
    def _copy_tail():
        rows = _N - (_TILES - 1) * _ROWS
        pltpu.sync_copy(blk_v.at[pl.ds(0, rows * _N)],
                        out_hbm.at[pl.ds(r0 * _N, rows * _N)])


@functools.cache
def _build_adj_kernel():
    return pl.kernel(
        _adj_body,
        out_type=jax.ShapeDtypeStruct((_N * _N,), jnp.float32),
        mesh=plsc.VectorSubcoreMesh(core_axis_name="c", subcore_axis_name="s"),
        compiler_params=pltpu.CompilerParams(needs_layout_passes=False),
        scratch_types=[
            pltpu.VMEM((_E,), jnp.int32),
            pltpu.VMEM((_E,), jnp.int32),
            pltpu.VMEM((_NPAD,), jnp.float32),
            pltpu.VMEM((16 * _DPS,), jnp.float32),
            pltpu.VMEM((_NPAD,), jnp.float32),
            pltpu.VMEM((_DPS,), jnp.float32),
            pltpu.VMEM((_ROWS * _N,), jnp.float32),
            pltpu.VMEM_SHARED((16 * _NPAD,), jnp.float32),
            pltpu.VMEM_SHARED((_NPAD,), jnp.float32),
        ],
    )


def _gcn_body(xt_ref, A_ref, wk_ref, mavg_ref, vec_ref, out_ref):
    A = A_ref[...]
    M = mavg_ref[...]
    halves = [xt_ref[:, k * _COLS:(k + 1) * _COLS] for k in range(_HALVES)]
    for li in range(3):
        wk = wk_ref[li]
        bias = vec_ref[3 * li:3 * li + 1, :]
        g = vec_ref[3 * li + 1:3 * li + 2, :]
        b = vec_ref[3 * li + 2:3 * li + 3, :]
        hl = jnp.concatenate(
            [jax.lax.dot(h, wk, preferred_element_type=jnp.float32)
             for h in halves], axis=1)
        ha_w = jax.lax.dot(A, hl, preferred_element_type=jnp.float32)
        nxt = []
        for k in range(_HALVES):
            ha = ha_w[:, k * _COLS:(k + 1) * _COLS] + bias
            m = jax.lax.dot(ha, M, preferred_element_type=jnp.float32)
            q = jax.lax.dot(ha * ha, M, preferred_element_type=jnp.float32)
            o = jnp.maximum((ha - m) * lax.rsqrt(q - m * m + 1e-5) * g + b,
                            0.0)
            nxt.append(o if li == 0 else o + halves[k])
        halves = nxt
    out = jnp.concatenate(
        [jnp.mean(h, axis=0, keepdims=True) for h in halves], axis=1)
    out_ref[...] = out.reshape(1, 1, _WIDE)


def _tail_body(seq_ref, P_ref, w1_ref, b1_ref, w2_ref, b2_ref, out_ref):
    pooled = jax.lax.dot(P_ref[...], seq_ref[...],
                         preferred_element_type=jnp.float32)
    hm = jnp.maximum(jax.lax.dot(pooled, w1_ref[...],
                                 preferred_element_type=jnp.float32)
                     + b1_ref[...], 0.0)
    out_ref[...] = jax.lax.dot(hm, w2_ref[...],
                               preferred_element_type=jnp.float32) + b2_ref[...]


def _block_diag(W, k):
    n, m = W.shape
    out = jnp.zeros((k * n, k * m), W.dtype)
    for i in range(k):
        out = out.at[i * n:(i + 1) * n, i * m:(i + 1) * m].set(W)
    return out


def kernel(x, base_adj, base_edge_index, W_in, b_in, ln_in_g, ln_in_b,
           W_h0, b_h0, ln_h0_g, ln_h0_b, W_h1, b_h1, ln_h1_g, ln_h1_b,
           W_c1, b_c1, W_c2, b_c2):
    del base_adj
    src = base_edge_index[0]
    dst = base_edge_index[1]
    A = _build_adj_kernel()(src, dst).reshape(_N, _N)

    xt = jnp.transpose(x.reshape(_NG, _N, _H), (1, 0, 2)).reshape(_N, _NG * _H)

    wks = jnp.stack([_block_diag(W_in, _GPB), _block_diag(W_h0, _GPB),
                     _block_diag(W_h1, _GPB)])
    mavg = _block_diag(jnp.full((_H, _H), 1.0 / _H, jnp.float32), _GPB)
    vecs = jnp.zeros((16, _COLS), jnp.float32)
    for r_i, v in enumerate([b_in, ln_in_g, ln_in_b, b_h0, ln_h0_g, ln_h0_b,
                             b_h1, ln_h1_g, ln_h1_b]):
        vecs = vecs.at[r_i].set(jnp.tile(v, _GPB))

    seq = pl.pallas_call(
        _gcn_body,
        grid=(_STEPS,),
        in_specs=[
            pl.BlockSpec((_N, _WIDE), lambda i: (0, i)),
            pl.BlockSpec((_N, _N), lambda i: (0, 0)),
            pl.BlockSpec((3, _COLS, _COLS), lambda i: (0, 0, 0)),
            pl.BlockSpec((_COLS, _COLS), lambda i: (0, 0)),
            pl.BlockSpec((16, _COLS), lambda i: (0, 0)),
        ],
        out_specs=pl.BlockSpec((1, 1, _WIDE), lambda i: (i, 0, 0)),
        out_shape=jax.ShapeDtypeStruct((_STEPS, 1, _WIDE), jnp.float32),
        compiler_params=pltpu.CompilerParams(
            dimension_semantics=("arbitrary",)),
    )(xt, A, wks, mavg, vecs)

    seq = seq.reshape(_NG, _H)
    P = jnp.repeat(jnp.eye(8, dtype=jnp.float32), 8, axis=1) / 8.0
    logits = pl.pallas_call(
        _tail_body,
        out_shape=jax.ShapeDtypeStruct((8, 2), jnp.float32),
    )(seq, P, W_c1, b_c1.reshape(1, _H), W_c2, b_c2.reshape(1, 2))
    return logits
